# baseline (speedup 1.0000x reference)
import jax
import jax.numpy as jnp
from jax import lax
from jax.experimental import pallas as pl
from jax.experimental.pallas import tpu as pltpu
from jax.experimental.pallas import tpu_sc as plsc

VOCAB = 1000000
D_MODEL = 32
BATCH = 16384
HIST = 50

NC = 2
NS = 16
NW = NC * NS

SCOLS = 512
COLS_PER_W = 31232
T_STAGES = COLS_PER_W // SCOLS
EXTRA0 = COLS_PER_W * NW
TAIL0 = EXTRA0 + SCOLS
TAIL_COLS = VOCAB - TAIL0
IN_PITCH = SCOLS + 3

ROWS_PER_W = BATCH // NW
STAGE_ROWS = 16
N_STAGES = ROWS_PER_W // STAGE_ROWS


def _transpose_body(tab_hbm, tail_hbm, out_hbm, in_a, in_b, out_a, out_b,
                    tail_v, isems, osems):
    wid = lax.axis_index("s") * NC + lax.axis_index("c")
    col0 = wid * COLS_PER_W
    w_lo = lax.iota(jnp.int32, 16)
    w_hi = w_lo + 16
    in_bufs = (in_a, in_b)
    out_bufs = (out_a, out_b)

    def issue_in(t, parity):
        pltpu.async_copy(
            tab_hbm.at[:, pl.ds(col0 + t * SCOLS, SCOLS)],
            in_bufs[parity].at[:, pl.ds(0, SCOLS)],
            isems[parity],
        )

    def wait_in(parity):
        pltpu.make_async_copy(
            tab_hbm.at[:, pl.ds(0, SCOLS)],
            in_bufs[parity].at[:, pl.ds(0, SCOLS)],
            isems[parity],
        ).wait()

    def drain_out(parity):
        pltpu.make_async_copy(
            out_hbm.at[pl.ds(0, SCOLS * D_MODEL)],
            out_bufs[parity],
            osems[parity],
        ).wait()

    def transpose(src, dst, n):
        @plsc.parallel_loop(0, n, unroll=64, carry=jnp.zeros((16,), jnp.int32))
        def _tr(i, iv):
            a = plsc.load_gather(src, [w_lo, iv])
            b = plsc.load_gather(src, [w_hi, iv])
            dst[pl.ds(i * D_MODEL, 16)] = a
            dst[pl.ds(i * D_MODEL + 16, 16)] = b
            return iv + 1

    def store(t, parity):
        pltpu.async_copy(
            out_bufs[parity],
            out_hbm.at[pl.ds((col0 + t * SCOLS) * D_MODEL, SCOLS * D_MODEL)],
            osems[parity],
        )

    def stage(t, parity, with_drain, with_issue):
        wait_in(parity)
        if with_drain:
            drain_out(parity)
        transpose(in_bufs[parity], out_bufs[parity], SCOLS)
        store(t, parity)
        if with_issue:
            issue_in(t + 2, parity)

    issue_in(0, 0)
    issue_in(1, 1)
    stage(0, 0, False, True)
    stage(1, 1, False, True)

    def step(k, carry):
        del carry
        stage(2 * k, 0, True, True)
        stage(2 * k + 1, 1, True, True)
        return 0

    lax.fori_loop(1, (T_STAGES - 3) // 2, step, 0, unroll=False)
    stage(T_STAGES - 3, 0, True, True)
    stage(T_STAGES - 2, 1, True, False)
    stage(T_STAGES - 1, 0, True, False)
    drain_out(0)
    drain_out(1)

    @pl.when(wid == 0)
    def _extra():
        pltpu.async_copy(
            tab_hbm.at[:, pl.ds(EXTRA0, SCOLS)],
            in_a.at[:, pl.ds(0, SCOLS)],
            isems[0],
        ).wait()
        transpose(in_a, out_a, SCOLS)
        pltpu.async_copy(
            out_a,
            out_hbm.at[pl.ds(EXTRA0 * D_MODEL, SCOLS * D_MODEL)],
            osems[0],
        ).wait()

    @pl.when(wid == 4)
    def _tail():
        pltpu.sync_copy(tail_hbm, tail_v)

        @plsc.parallel_loop(0, TAIL_COLS, unroll=8)
        def _tr(i):
            iv = jnp.broadcast_to(i, (16,)).astype(jnp.int32)
            a = plsc.load_gather(tail_v, [w_lo, iv])
            b = plsc.load_gather(tail_v, [w_hi, iv])
            out_a[pl.ds(i * D_MODEL, 16)] = a
            out_a[pl.ds(i * D_MODEL + 16, 16)] = b
        pltpu.async_copy(
            out_a.at[pl.ds(0, TAIL_COLS * D_MODEL)],
            out_hbm.at[pl.ds(TAIL0 * D_MODEL, TAIL_COLS * D_MODEL)],
            osems[0],
        ).wait()


def _gather_body(idx_hbm, table_hbm, out_hbm, idx2d_v, stage_v, gsems, osems):
    wid = lax.axis_index("s") * NC + lax.axis_index("c")
    b0 = wid * ROWS_PER_W
    pltpu.sync_copy(idx_hbm.at[pl.ds(b0, ROWS_PER_W), :], idx2d_v)

    def issue_gathers(t, parity):
        for r in range(STAGE_ROWS):
            pltpu.async_copy(
                table_hbm.at[idx2d_v.at[t * STAGE_ROWS + r]],
                stage_v.at[parity * STAGE_ROWS + r],
                gsems[parity],
            )

    def drain(sem, parity):
        pltpu.make_async_copy(
            out_hbm.at[pl.ds(0, STAGE_ROWS)],
            stage_v.at[pl.ds(parity * STAGE_ROWS, STAGE_ROWS)],
            sem,
        ).wait()

    def store(t, parity):
        pltpu.async_copy(
            stage_v.at[pl.ds(parity * STAGE_ROWS, STAGE_ROWS)],
            out_hbm.at[pl.ds(b0 + t * STAGE_ROWS, STAGE_ROWS)],
            osems[parity],
        )

    issue_gathers(0, 0)
    issue_gathers(1, 1)
    drain(gsems[0], 0)
    store(0, 0)

    def step(k, carry):
        del carry
        t0 = 2 * k
        drain(osems[0], 0)
        issue_gathers(t0, 0)
        drain(gsems[1], 1)
        store(t0 - 1, 1)
        t1 = t0 + 1
        drain(osems[1], 1)
        issue_gathers(t1, 1)
        drain(gsems[0], 0)
        store(t1 - 1, 0)
        return 0

    lax.fori_loop(1, N_STAGES // 2, step, 0, unroll=False)
    drain(gsems[1], 1)
    store(N_STAGES - 1, 1)
    drain(osems[0], 0)
    drain(osems[1], 1)


def _sc_mesh():
    return plsc.VectorSubcoreMesh(
        core_axis_name="c", subcore_axis_name="s", num_cores=NC, num_subcores=NS
    )


@jax.jit
def _embed(idx, table):
    table_cm = table.T
    tail = lax.slice(table_cm, (0, TAIL0), (D_MODEL, VOCAB))
    table_flat = pl.kernel(
        _transpose_body,
        out_type=jax.ShapeDtypeStruct((VOCAB * D_MODEL,), jnp.float32),
        mesh=_sc_mesh(),
        compiler_params=pltpu.CompilerParams(needs_layout_passes=False),
        scratch_types=[
            pltpu.VMEM((D_MODEL, IN_PITCH), jnp.float32),
            pltpu.VMEM((D_MODEL, IN_PITCH), jnp.float32),
            pltpu.VMEM((SCOLS * D_MODEL,), jnp.float32),
            pltpu.VMEM((SCOLS * D_MODEL,), jnp.float32),
            pltpu.VMEM((D_MODEL, TAIL_COLS), jnp.float32),
            (pltpu.SemaphoreType.DMA, pltpu.SemaphoreType.DMA),
            (pltpu.SemaphoreType.DMA, pltpu.SemaphoreType.DMA),
        ],
    )(table_cm, tail)
    table_rm = table_flat.reshape(VOCAB, D_MODEL)
    return pl.kernel(
        _gather_body,
        out_type=jax.ShapeDtypeStruct((BATCH, HIST, D_MODEL), jnp.float32),
        mesh=_sc_mesh(),
        compiler_params=pltpu.CompilerParams(use_tc_tiling_on_sc=False),
        scratch_types=[
            pltpu.VMEM((ROWS_PER_W, HIST), jnp.int32),
            pltpu.VMEM((2 * STAGE_ROWS, HIST, D_MODEL), jnp.float32),
            (pltpu.SemaphoreType.DMA, pltpu.SemaphoreType.DMA),
            (pltpu.SemaphoreType.DMA, pltpu.SemaphoreType.DMA),
        ],
    )(idx, table_rm)


def kernel(indice_sequence, embedding_matrix):
    return _embed(indice_sequence.astype(jnp.int32), embedding_matrix)


# speedup vs baseline: 1.0636x; 1.0636x over previous
import jax
import jax.numpy as jnp
from jax import lax
from jax.experimental import pallas as pl
from jax.experimental.pallas import tpu as pltpu
from jax.experimental.pallas import tpu_sc as plsc

VOCAB = 1000000
D_MODEL = 32
BATCH = 16384
HIST = 50

NC = 2
NS = 16
NW = NC * NS

SCOLS = 512
COLS_PER_W = 31232
T_STAGES = COLS_PER_W // SCOLS
EXTRA0 = COLS_PER_W * NW
TAIL0 = EXTRA0 + SCOLS
TAIL_COLS = VOCAB - TAIL0
IN_PITCH = SCOLS + 3

ROWS_PER_W = BATCH // NW
STAGE_ROWS = 16
N_STAGES = ROWS_PER_W // STAGE_ROWS


def _transpose_body(tab_hbm, tail_hbm, out_hbm, in_a, in_b, out_a, out_b,
                    tail_v, isems, osems):
    wid = lax.axis_index("s") * NC + lax.axis_index("c")
    col0 = wid * COLS_PER_W
    w_lo = lax.iota(jnp.int32, 16)
    w_hi = w_lo + 16
    in_bufs = (in_a, in_b)
    out_bufs = (out_a, out_b)

    def issue_in(t, parity):
        pltpu.async_copy(
            tab_hbm.at[:, pl.ds(col0 + t * SCOLS, SCOLS)],
            in_bufs[parity].at[:, pl.ds(0, SCOLS)],
            isems[parity],
        )

    def wait_in(parity):
        pltpu.make_async_copy(
            tab_hbm.at[:, pl.ds(0, SCOLS)],
            in_bufs[parity].at[:, pl.ds(0, SCOLS)],
            isems[parity],
        ).wait()

    def drain_out(parity):
        pltpu.make_async_copy(
            out_hbm.at[pl.ds(0, SCOLS * D_MODEL)],
            out_bufs[parity],
            osems[parity],
        ).wait()

    def transpose(src, dst, n):
        @plsc.parallel_loop(0, n, unroll=32, carry=jnp.zeros((16,), jnp.int32))
        def _tr(i, iv):
            a = plsc.load_gather(src, [w_lo, iv])
            b = plsc.load_gather(src, [w_hi, iv])
            dst[pl.ds(i * D_MODEL, 16)] = a
            dst[pl.ds(i * D_MODEL + 16, 16)] = b
            return iv + 1

    def store(t, parity):
        pltpu.async_copy(
            out_bufs[parity],
            out_hbm.at[pl.ds((col0 + t * SCOLS) * D_MODEL, SCOLS * D_MODEL)],
            osems[parity],
        )

    def stage(t, parity, with_drain, with_issue):
        wait_in(parity)
        if with_drain:
            drain_out(parity)
        transpose(in_bufs[parity], out_bufs[parity], SCOLS)
        store(t, parity)
        if with_issue:
            issue_in(t + 2, parity)

    issue_in(0, 0)
    issue_in(1, 1)
    stage(0, 0, False, True)
    stage(1, 1, False, True)

    def step(k, carry):
        del carry
        stage(2 * k, 0, True, True)
        stage(2 * k + 1, 1, True, True)
        return 0

    lax.fori_loop(1, (T_STAGES - 3) // 2, step, 0, unroll=False)
    stage(T_STAGES - 3, 0, True, True)
    stage(T_STAGES - 2, 1, True, False)
    stage(T_STAGES - 1, 0, True, False)
    drain_out(0)
    drain_out(1)

    @pl.when(wid == 0)
    def _extra():
        pltpu.async_copy(
            tab_hbm.at[:, pl.ds(EXTRA0, SCOLS)],
            in_a.at[:, pl.ds(0, SCOLS)],
            isems[0],
        ).wait()
        transpose(in_a, out_a, SCOLS)
        pltpu.async_copy(
            out_a,
            out_hbm.at[pl.ds(EXTRA0 * D_MODEL, SCOLS * D_MODEL)],
            osems[0],
        ).wait()

    @pl.when(wid == 4)
    def _tail():
        pltpu.sync_copy(tail_hbm, tail_v)

        @plsc.parallel_loop(0, TAIL_COLS, unroll=8)
        def _tr(i):
            iv = jnp.broadcast_to(i, (16,)).astype(jnp.int32)
            a = plsc.load_gather(tail_v, [w_lo, iv])
            b = plsc.load_gather(tail_v, [w_hi, iv])
            out_a[pl.ds(i * D_MODEL, 16)] = a
            out_a[pl.ds(i * D_MODEL + 16, 16)] = b
        pltpu.async_copy(
            out_a.at[pl.ds(0, TAIL_COLS * D_MODEL)],
            out_hbm.at[pl.ds(TAIL0 * D_MODEL, TAIL_COLS * D_MODEL)],
            osems[0],
        ).wait()


def _gather_body(idx_hbm, table_hbm, out_hbm, idx2d_v, stage_v, gsems, osems):
    wid = lax.axis_index("s") * NC + lax.axis_index("c")
    b0 = wid * ROWS_PER_W
    pltpu.sync_copy(idx_hbm.at[pl.ds(b0, ROWS_PER_W), :], idx2d_v)

    def issue_gathers(t, parity):
        for r in range(STAGE_ROWS):
            pltpu.async_copy(
                table_hbm.at[idx2d_v.at[t * STAGE_ROWS + r]],
                stage_v.at[parity * STAGE_ROWS + r],
                gsems[parity],
            )

    def drain(sem, parity):
        pltpu.make_async_copy(
            out_hbm.at[pl.ds(0, STAGE_ROWS)],
            stage_v.at[pl.ds(parity * STAGE_ROWS, STAGE_ROWS)],
            sem,
        ).wait()

    def store(t, parity):
        pltpu.async_copy(
            stage_v.at[pl.ds(parity * STAGE_ROWS, STAGE_ROWS)],
            out_hbm.at[pl.ds(b0 + t * STAGE_ROWS, STAGE_ROWS)],
            osems[parity],
        )

    issue_gathers(0, 0)
    issue_gathers(1, 1)
    drain(gsems[0], 0)
    store(0, 0)

    def step(k, carry):
        del carry
        t0 = 2 * k
        drain(osems[0], 0)
        issue_gathers(t0, 0)
        drain(gsems[1], 1)
        store(t0 - 1, 1)
        t1 = t0 + 1
        drain(osems[1], 1)
        issue_gathers(t1, 1)
        drain(gsems[0], 0)
        store(t1 - 1, 0)
        return 0

    lax.fori_loop(1, N_STAGES // 2, step, 0, unroll=False)
    drain(gsems[1], 1)
    store(N_STAGES - 1, 1)
    drain(osems[0], 0)
    drain(osems[1], 1)


def _sc_mesh():
    return plsc.VectorSubcoreMesh(
        core_axis_name="c", subcore_axis_name="s", num_cores=NC, num_subcores=NS
    )


@jax.jit
def _embed(idx, table):
    table_cm = table.T
    tail = lax.slice(table_cm, (0, TAIL0), (D_MODEL, VOCAB))
    table_flat = pl.kernel(
        _transpose_body,
        out_type=jax.ShapeDtypeStruct((VOCAB * D_MODEL,), jnp.float32),
        mesh=_sc_mesh(),
        compiler_params=pltpu.CompilerParams(needs_layout_passes=False),
        scratch_types=[
            pltpu.VMEM((D_MODEL, IN_PITCH), jnp.float32),
            pltpu.VMEM((D_MODEL, IN_PITCH), jnp.float32),
            pltpu.VMEM((SCOLS * D_MODEL,), jnp.float32),
            pltpu.VMEM((SCOLS * D_MODEL,), jnp.float32),
            pltpu.VMEM((D_MODEL, TAIL_COLS), jnp.float32),
            (pltpu.SemaphoreType.DMA, pltpu.SemaphoreType.DMA),
            (pltpu.SemaphoreType.DMA, pltpu.SemaphoreType.DMA),
        ],
    )(table_cm, tail)
    table_rm = table_flat.reshape(VOCAB, D_MODEL)
    return pl.kernel(
        _gather_body,
        out_type=jax.ShapeDtypeStruct((BATCH, HIST, D_MODEL), jnp.float32),
        mesh=_sc_mesh(),
        compiler_params=pltpu.CompilerParams(use_tc_tiling_on_sc=False),
        scratch_types=[
            pltpu.VMEM((ROWS_PER_W, HIST), jnp.int32),
            pltpu.VMEM((2 * STAGE_ROWS, HIST, D_MODEL), jnp.float32),
            (pltpu.SemaphoreType.DMA, pltpu.SemaphoreType.DMA),
            (pltpu.SemaphoreType.DMA, pltpu.SemaphoreType.DMA),
        ],
    )(idx, table_rm)


def kernel(indice_sequence, embedding_matrix):
    return _embed(indice_sequence.astype(jnp.int32), embedding_matrix)
